# Initial kernel scaffold; baseline (speedup 1.0000x reference)
import jax
import jax.numpy as jnp
from jax.experimental import pallas as pl


def kernel(x, Wq, keys, values, bn_gamma, bn_beta):
    raise NotImplementedError("write your pallas kernel here")


# speedup vs baseline: 3.2360x; 3.2360x over previous
import functools

import jax
import jax.numpy as jnp
from jax import lax
from jax.experimental import pallas as pl
from jax.experimental.pallas import tpu as pltpu
from jax.experimental.pallas import tpu_sc as plsc

B, T, DIM = 1, 2048, 1024
HEADS, NUM_KEYS, TOPK = 4, 256, 32
D2 = DIM // HEADS // 2
NSLOT = HEADS * 2
NG = HEADS
PAIR = TOPK * TOPK
ROWS_PER_TOK = NG * TOPK

T_TILE = 256
N_TILES = T // T_TILE

NEG = -1e30


def _qstats_body(x_ref, wq_ref, q_ref, psum_ref, psq_ref):
    q = lax.dot_general(x_ref[...], wq_ref[...],
                        (((1,), (1,)), ((), ())),
                        preferred_element_type=jnp.float32)
    q_ref[...] = q
    psum_ref[...] = jnp.sum(q, axis=0, keepdims=True)[None]
    psq_ref[...] = jnp.sum(q * q, axis=0, keepdims=True)[None]


def _q_and_stats(x2d, wq):
    return pl.pallas_call(
        _qstats_body,
        grid=(N_TILES,),
        in_specs=[
            pl.BlockSpec((T_TILE, DIM), lambda i: (i, 0)),
            pl.BlockSpec((DIM, DIM), lambda i: (0, 0)),
        ],
        out_specs=[
            pl.BlockSpec((T_TILE, DIM), lambda i: (i, 0)),
            pl.BlockSpec((1, 1, DIM), lambda i: (i, 0, 0)),
            pl.BlockSpec((1, 1, DIM), lambda i: (i, 0, 0)),
        ],
        out_shape=[
            jax.ShapeDtypeStruct((T, DIM), jnp.float32),
            jax.ShapeDtypeStruct((N_TILES, 1, DIM), jnp.float32),
            jax.ShapeDtypeStruct((N_TILES, 1, DIM), jnp.float32),
        ],
    )(x2d, wq)


_CL = [32, 16, 10, 8, 6, 5, 4, 4, 3, 3, 2, 2, 2, 2, 2, 2]
NCAND = 128
_NREAL = sum(_CL) + 16

_IMIN = -2147483648


def _mono(f):
    b = lax.bitcast_convert_type(f, jnp.int32)
    return b ^ ((b >> 31) & jnp.int32(0x7FFFFFFF))


def _demono(m):
    b = m ^ ((m >> 31) & jnp.int32(0x7FFFFFFF))
    return lax.bitcast_convert_type(b, jnp.float32)


def _topk_packed(kp, iota_k):
    m0 = jnp.max(kp, axis=-1, keepdims=True)
    lst0 = jnp.where(iota_k == 0, m0,
                     jnp.zeros(kp.shape[:-1] + (TOPK,), jnp.int32))

    def body(k, carry):
        m, lst = carry
        z = jnp.where(kp < m, kp, jnp.int32(_IMIN))
        m = jnp.max(z, axis=-1, keepdims=True)
        lst = jnp.where(iota_k == k, m, lst)
        return m, lst

    _, lst = lax.fori_loop(1, TOPK, body, (m0, lst0))
    return lst


def _route_body(q_ref, psum_ref, psq_ref, gamma_ref, beta_ref, keys_ref,
                attn_ref, cid_ref, icand_ref):
    mean = jnp.sum(psum_ref[...], axis=0) / float(T)
    ex2 = jnp.sum(psq_ref[...], axis=0) / float(T)
    var = ex2 - mean * mean
    a = gamma_ref[...] * lax.rsqrt(var + 1e-5)
    b = beta_ref[...] - mean * a
    qn = q_ref[...] * a + b

    iota_n = lax.broadcasted_iota(jnp.int32, (T_TILE, NSLOT, NUM_KEYS), 2)
    iota_k = lax.broadcasted_iota(jnp.int32, (T_TILE, NSLOT, TOPK), 2)
    iota_k4 = lax.broadcasted_iota(jnp.int32, (T_TILE, NG, TOPK), 2)
    iota_c = lax.broadcasted_iota(jnp.int32, (T_TILE, NG, NCAND), 2)

    ds = []
    for s in range(NSLOT):
        h, p = s // 2, s % 2
        col = p * (DIM // 2) + h * D2
        d = lax.dot_general(qn[:, col:col + D2], keys_ref[s],
                            (((1,), (0,)), ((), ())),
                            preferred_element_type=jnp.float32)
        ds.append(d[:, None, :])
    dall = jnp.concatenate(ds, axis=1)
    kp = (_mono(dall) & jnp.int32(~255)) | iota_n
    lst = _topk_packed(kp, iota_k)
    vals = _demono(lst & jnp.int32(~255))
    idxs = lst & jnp.int32(255)

    s0, s1 = vals[:, :NG, :], vals[:, NG:, :]
    i0, i1 = idxs[:, :NG, :], idxs[:, NG:, :]
    scols, icols = [], []
    for i in range(16):
        scols.append(s0[:, :, i:i + 1] + s1[:, :, :_CL[i]])
        icols.append(i0[:, :, i:i + 1] * NUM_KEYS + i1[:, :, :_CL[i]])
    scols.append(s0[:, :, 16:32] + s1[:, :, 0:1])
    icols.append(i0[:, :, 16:32] * NUM_KEYS + i1[:, :, 0:1])
    scols.append(jnp.full((T_TILE, NG, NCAND - _NREAL), -3e38, jnp.float32))
    icols.append(jnp.zeros((T_TILE, NG, NCAND - _NREAL), jnp.int32))
    sc = jnp.concatenate(scols, axis=2)
    ic = jnp.concatenate(icols, axis=2)
    icand_ref[...] = ic

    kp2 = (_mono(sc) & jnp.int32(~127)) | iota_c
    lst2 = _topk_packed(kp2, iota_k4)
    cid_ref[...] = lst2 & jnp.int32(127)
    fv = _demono(lst2 & jnp.int32(~127))
    e = jnp.exp(fv - fv[:, :, 0:1])
    attn_ref[...] = e / jnp.sum(e, axis=2, keepdims=True)


def _route(q, psum, psq, gamma, beta, keys_r):
    ntok = q.shape[0]
    return pl.pallas_call(
        _route_body,
        grid=(ntok // T_TILE,),
        in_specs=[
            pl.BlockSpec((T_TILE, DIM), lambda i: (i, 0)),
            pl.BlockSpec((N_TILES, 1, DIM), lambda i: (0, 0, 0)),
            pl.BlockSpec((N_TILES, 1, DIM), lambda i: (0, 0, 0)),
            pl.BlockSpec((1, DIM), lambda i: (0, 0)),
            pl.BlockSpec((1, DIM), lambda i: (0, 0)),
            pl.BlockSpec((NSLOT, D2, NUM_KEYS), lambda i: (0, 0, 0)),
        ],
        out_specs=[
            pl.BlockSpec((T_TILE, NG, TOPK), lambda i: (i, 0, 0)),
            pl.BlockSpec((T_TILE, NG, TOPK), lambda i: (i, 0, 0)),
            pl.BlockSpec((T_TILE, NG, NCAND), lambda i: (i, 0, 0)),
        ],
        out_shape=[
            jax.ShapeDtypeStruct((ntok, NG, TOPK), jnp.float32),
            jax.ShapeDtypeStruct((ntok, NG, TOPK), jnp.int32),
            jax.ShapeDtypeStruct((ntok, NG, NCAND), jnp.int32),
        ],
    )(q, psum, psq, gamma, beta, keys_r)


_SC_INFO = plsc.get_sparse_core_info()
_NW = _SC_INFO.num_cores * _SC_INFO.num_subcores
TOK_PER_W = T // _NW
CHUNK = 32
NCHUNK = ROWS_PER_TOK // CHUNK
FBLK = 16
NFP = DIM // (FBLK * 16)


def _make_bag_body(tok_per_w):
    return functools.partial(_bag_body_impl, tok_per_w)


def _bag_body_impl(tok_per_w, vals_hbm, cid_hbm, icand_hbm, w_hbm, out_hbm,
                   cid_v, icand_v, w_v, vidx_v, rows0, rows1, acc,
                   sem0, sem1, sem_m):
    wid = lax.axis_index("c") * _SC_INFO.num_subcores + lax.axis_index("s")
    base = wid * tok_per_w
    pltpu.sync_copy(w_hbm.at[pl.ds(base, tok_per_w)], w_v)
    pltpu.sync_copy(cid_hbm.at[pl.ds(base, 1)], cid_v)
    pltpu.sync_copy(icand_hbm.at[pl.ds(base, 1)], icand_v)

    rows = (rows0, rows1)
    sems = (sem0, sem1)
    zs = jnp.zeros((16,), jnp.int32)

    def token(t, _):
        for g in range(NG):
            gs = jnp.full((16,), g, jnp.int32)
            for hh in range(2):
                lanes = cid_v[0, g, pl.ds(hh * 16, 16)]
                got = plsc.load_gather(icand_v, [zs, gs, lanes])
                vidx_v[g, pl.ds(hh * 16, 16)] = got

        tn = base + jnp.minimum(t + 1, tok_per_w - 1)
        mc = pltpu.make_async_copy(cid_hbm.at[pl.ds(tn, 1)], cid_v, sem_m)
        mi = pltpu.make_async_copy(icand_hbm.at[pl.ds(tn, 1)], icand_v, sem_m)
        mc.start()
        mi.start()

        cp0 = pltpu.make_async_copy(vals_hbm.at[vidx_v.at[0]], rows0, sem0)
        cp0.start()
        for c in range(NCHUNK):
            if c + 1 < NCHUNK:
                nxt = pltpu.make_async_copy(
                    vals_hbm.at[vidx_v.at[c + 1]],
                    rows[(c + 1) % 2], sems[(c + 1) % 2])
                nxt.start()
            pltpu.make_async_copy(
                vals_hbm.at[vidx_v.at[c]],
                rows[c % 2], sems[c % 2]).wait()
            buf = rows[c % 2]
            lane = lax.iota(jnp.int32, 16)

            def wscal(t_, r):
                w16 = w_v[t_, pl.ds(c * CHUNK + (r // 16) * 16, 16)]
                return jnp.sum(jnp.where(lane == r % 16, w16, 0.0))

            for f in range(NFP):
                def frow(r, carry):
                    w = wscal(t, r)
                    return tuple(
                        carry[v] + w * buf[r, pl.ds(f * FBLK * 16 + v * 16, 16)]
                        for v in range(FBLK))
                if c == 0:
                    w0 = wscal(t, jnp.int32(0))
                    init = tuple(
                        w0 * buf[0, pl.ds(f * FBLK * 16 + v * 16, 16)]
                        for v in range(FBLK))
                    res = lax.fori_loop(1, CHUNK, frow, init)
                else:
                    init = tuple(
                        acc[0, pl.ds(f * FBLK * 16 + v * 16, 16)]
                        for v in range(FBLK))
                    res = lax.fori_loop(0, CHUNK, frow, init)
                for v in range(FBLK):
                    acc[0, pl.ds(f * FBLK * 16 + v * 16, 16)] = res[v]
        pltpu.sync_copy(acc, out_hbm.at[pl.ds(base + t, 1)])
        mc.wait()
        mi.wait()
        return 0

    lax.fori_loop(0, tok_per_w, token, 0)


def _bag(values, cid, icand, attn):
    ntok = cid.shape[0]
    tok_per_w = ntok // _NW
    mesh = plsc.VectorSubcoreMesh(core_axis_name="c", subcore_axis_name="s")
    fn = pl.kernel(
        _make_bag_body(tok_per_w),
        mesh=mesh,
        compiler_params=pltpu.CompilerParams(needs_layout_passes=False),
        out_type=jax.ShapeDtypeStruct((ntok, DIM), jnp.float32),
        scratch_types=[
            pltpu.VMEM((1, NG, TOPK), jnp.int32),
            pltpu.VMEM((1, NG, NCAND), jnp.int32),
            pltpu.VMEM((tok_per_w, ROWS_PER_TOK), jnp.float32),
            pltpu.VMEM((NCHUNK, CHUNK), jnp.int32),
            pltpu.VMEM((CHUNK, DIM), jnp.float32),
            pltpu.VMEM((CHUNK, DIM), jnp.float32),
            pltpu.VMEM((1, DIM), jnp.float32),
            pltpu.SemaphoreType.DMA,
            pltpu.SemaphoreType.DMA,
            pltpu.SemaphoreType.DMA,
        ],
    )
    return fn(values, cid, icand, attn)


NSLICE = 8


@jax.jit
def kernel(x, Wq, keys, values, bn_gamma, bn_beta):
    x2d = x.reshape(T, DIM)
    q, psum, psq = _q_and_stats(x2d, Wq)
    keys_r = jnp.transpose(keys, (0, 2, 3, 1)).reshape(NSLOT, D2, NUM_KEYS)
    gamma = bn_gamma.reshape(1, DIM)
    beta = bn_beta.reshape(1, DIM)
    ts = T // NSLICE
    outs = []
    for sl in range(NSLICE):
        qs = lax.slice_in_dim(q, sl * ts, (sl + 1) * ts, axis=0)
        attn, cid, icand = _route(qs, psum, psq, gamma, beta, keys_r)
        outs.append(_bag(values, cid, icand, attn.reshape(ts, ROWS_PER_TOK)))
    out = jnp.concatenate(outs, axis=0)
    return out.reshape(B, T, DIM)
